# Initial kernel scaffold; baseline (speedup 1.0000x reference)
import jax
import jax.numpy as jnp
from jax.experimental import pallas as pl


def kernel(x, type_idx, type_embedding):
    raise NotImplementedError("write your pallas kernel here")


# speedup vs baseline: 2.6965x; 2.6965x over previous
import jax
import jax.numpy as jnp
from jax.experimental import pallas as pl
from jax.experimental.pallas import tpu as pltpu

D_MODEL = 1024
ROWS_PER_BLOCK = 512


def _body(idx_ref, x_ref, tab_ref, out_ref):
    idx = idx_ref[0, 0]
    idxc = idx[:, None]
    t0 = tab_ref[0][None, :]
    t1 = tab_ref[1][None, :]
    t2 = tab_ref[2][None, :]
    emb = jnp.where(idxc == 0, t0, jnp.where(idxc == 1, t1, t2))
    out_ref[...] = x_ref[...] + emb


def kernel(x, type_idx, type_embedding):
    B, S, D = x.shape
    N = B * S
    nblk = N // ROWS_PER_BLOCK
    x2 = x.reshape(nblk, ROWS_PER_BLOCK, D)
    idx2 = type_idx.reshape(nblk, 1, ROWS_PER_BLOCK).astype(jnp.int32)

    out = pl.pallas_call(
        _body,
        grid=(nblk,),
        in_specs=[
            pl.BlockSpec((1, 1, ROWS_PER_BLOCK), lambda i: (i, 0, 0)),
            pl.BlockSpec((1, ROWS_PER_BLOCK, D), lambda i: (i, 0, 0)),
            pl.BlockSpec((3, D), lambda i: (0, 0)),
        ],
        out_specs=pl.BlockSpec((1, ROWS_PER_BLOCK, D), lambda i: (i, 0, 0)),
        out_shape=jax.ShapeDtypeStruct((nblk, ROWS_PER_BLOCK, D), x.dtype),
    )(idx2, x2, type_embedding)
    return out.reshape(B, S, D)
